# baseline (speedup 1.0000x reference)
import functools

import jax
import jax.numpy as jnp
from jax import lax
from jax.experimental import pallas as pl
from jax.experimental.pallas import tpu as pltpu
from jax.experimental.pallas import tpu_sc as plsc

DIM = 128
LANES = 16
CHUNK = 128


def _make_kernel(batch):
    info = plsc.get_sparse_core_info()
    nc, ns = info.num_cores, info.num_subcores
    nw = nc * ns
    bpw = batch // nw
    nchunks = bpw // CHUNK

    mesh = plsc.VectorSubcoreMesh(core_axis_name="c", subcore_axis_name="s")

    @functools.partial(
        pl.kernel,
        mesh=mesh,
        out_type=jax.ShapeDtypeStruct((batch,), jnp.float32),
        compiler_params=pltpu.CompilerParams(needs_layout_passes=False),
        scratch_types=[
            pltpu.VMEM((bpw,), jnp.int32),
            pltpu.VMEM((bpw,), jnp.int32),
            pltpu.VMEM((CHUNK, DIM), jnp.float32),
            pltpu.VMEM((CHUNK, DIM), jnp.float32),
            pltpu.VMEM((bpw,), jnp.float32),
            pltpu.SemaphoreType.DMA,
            pltpu.SemaphoreType.DMA,
        ],
    )
    def two_tower(u_idx_hbm, i_idx_hbm, user_hbm, item_hbm, out_hbm,
                  uidx_v, iidx_v, u_rows, i_rows, out_v, usem, isem):
        wid = lax.axis_index("s") * nc + lax.axis_index("c")
        base = wid * bpw
        pltpu.sync_copy(u_idx_hbm.at[pl.ds(base, bpw)], uidx_v)
        pltpu.sync_copy(i_idx_hbm.at[pl.ds(base, bpw)], iidx_v)

        lane_iota = lax.iota(jnp.int32, LANES)

        def chunk_body(c, _):
            cu = pltpu.async_copy(
                user_hbm.at[uidx_v.at[pl.ds(c * CHUNK, CHUNK)]], u_rows, usem)
            ci = pltpu.async_copy(
                item_hbm.at[iidx_v.at[pl.ds(c * CHUNK, CHUNK)]], i_rows, isem)
            cu.wait()
            ci.wait()

            def group_body(g, _):
                rows = g * LANES + lane_iota
                acc = jnp.zeros((LANES,), jnp.float32)
                for d in range(DIM):
                    col = jnp.full((LANES,), d, jnp.int32)
                    ug = plsc.load_gather(u_rows, [rows, col])
                    ig = plsc.load_gather(i_rows, [rows, col])
                    acc = acc + ug * ig
                out_v[pl.ds(c * CHUNK + g * LANES, LANES)] = acc
                return 0

            lax.fori_loop(0, CHUNK // LANES, group_body, 0)
            return 0

        lax.fori_loop(0, nchunks, chunk_body, 0)
        pltpu.sync_copy(out_v, out_hbm.at[pl.ds(base, bpw)])

    return two_tower


@jax.jit
def kernel(u_idx, i_idx, user_emb, item_emb):
    return _make_kernel(u_idx.shape[0])(u_idx, i_idx, user_emb, item_emb)


# speedup vs baseline: 1.9654x; 1.9654x over previous
import functools

import jax
import jax.numpy as jnp
from jax import lax
from jax.experimental import pallas as pl
from jax.experimental.pallas import tpu as pltpu
from jax.experimental.pallas import tpu_sc as plsc

DIM = 128
LANES = 16
CHUNK = 128


def _make_kernel(batch):
    info = plsc.get_sparse_core_info()
    nc, ns = info.num_cores, info.num_subcores
    nw = nc * ns
    bpw = batch // nw
    nchunks = bpw // CHUNK

    mesh = plsc.VectorSubcoreMesh(core_axis_name="c", subcore_axis_name="s")

    @functools.partial(
        pl.kernel,
        mesh=mesh,
        out_type=jax.ShapeDtypeStruct((batch,), jnp.float32),
        compiler_params=pltpu.CompilerParams(needs_layout_passes=False),
        scratch_types=[
            pltpu.VMEM((bpw,), jnp.int32),
            pltpu.VMEM((bpw,), jnp.int32),
            pltpu.VMEM((CHUNK, DIM), jnp.float32),
            pltpu.VMEM((CHUNK, DIM), jnp.float32),
            pltpu.VMEM((bpw,), jnp.float32),
            pltpu.SemaphoreType.DMA,
            pltpu.SemaphoreType.DMA,
        ],
    )
    def two_tower(u_idx_hbm, i_idx_hbm, user_hbm, item_hbm, out_hbm,
                  uidx_v, iidx_v, u_rows, i_rows, out_v, usem, isem):
        wid = lax.axis_index("s") * nc + lax.axis_index("c")
        base = wid * bpw
        pltpu.sync_copy(u_idx_hbm.at[pl.ds(base, bpw)], uidx_v)
        pltpu.sync_copy(i_idx_hbm.at[pl.ds(base, bpw)], iidx_v)

        lane_iota = lax.iota(jnp.int32, LANES)

        def chunk_body(c, _):
            cu = pltpu.async_copy(
                user_hbm.at[uidx_v.at[pl.ds(c * CHUNK, CHUNK)]], u_rows, usem)
            ci = pltpu.async_copy(
                item_hbm.at[iidx_v.at[pl.ds(c * CHUNK, CHUNK)]], i_rows, isem)
            cu.wait()
            ci.wait()

            def group_body(g, _):
                acc = jnp.zeros((LANES,), jnp.float32)
                for r in range(LANES):
                    row = g * LANES + r
                    p = u_rows[row, pl.ds(0, LANES)] * \
                        i_rows[row, pl.ds(0, LANES)]
                    for cc in range(1, DIM // LANES):
                        p = p + u_rows[row, pl.ds(cc * LANES, LANES)] * \
                                i_rows[row, pl.ds(cc * LANES, LANES)]
                    s = jnp.sum(p)
                    acc = jnp.where(lane_iota == r, s, acc)
                out_v[pl.ds(c * CHUNK + g * LANES, LANES)] = acc
                return 0

            lax.fori_loop(0, CHUNK // LANES, group_body, 0)
            return 0

        lax.fori_loop(0, nchunks, chunk_body, 0)
        pltpu.sync_copy(out_v, out_hbm.at[pl.ds(base, bpw)])

    return two_tower


@jax.jit
def kernel(u_idx, i_idx, user_emb, item_emb):
    return _make_kernel(u_idx.shape[0])(u_idx, i_idx, user_emb, item_emb)


# speedup vs baseline: 2.4894x; 1.2666x over previous
import functools

import jax
import jax.numpy as jnp
from jax import lax
from jax.experimental import pallas as pl
from jax.experimental.pallas import tpu as pltpu
from jax.experimental.pallas import tpu_sc as plsc

DIM = 128
LANES = 16
CHUNK = 128


def _make_kernel(batch):
    info = plsc.get_sparse_core_info()
    nc, ns = info.num_cores, info.num_subcores
    nw = nc * ns
    bpw = batch // nw
    nchunks = bpw // CHUNK

    mesh = plsc.VectorSubcoreMesh(core_axis_name="c", subcore_axis_name="s")

    @functools.partial(
        pl.kernel,
        mesh=mesh,
        out_type=jax.ShapeDtypeStruct((batch,), jnp.float32),
        compiler_params=pltpu.CompilerParams(needs_layout_passes=False),
        scratch_types=[
            pltpu.VMEM((bpw,), jnp.int32),
            pltpu.VMEM((bpw,), jnp.int32),
            pltpu.VMEM((2, CHUNK, DIM), jnp.float32),
            pltpu.VMEM((2, CHUNK, DIM), jnp.float32),
            pltpu.VMEM((bpw,), jnp.float32),
            pltpu.SemaphoreType.DMA,
            pltpu.SemaphoreType.DMA,
            pltpu.SemaphoreType.DMA,
            pltpu.SemaphoreType.DMA,
        ],
    )
    def two_tower(u_idx_hbm, i_idx_hbm, user_hbm, item_hbm, out_hbm,
                  uidx_v, iidx_v, u_rows, i_rows, out_v, *sems):
        wid = lax.axis_index("s") * nc + lax.axis_index("c")
        base = wid * bpw
        pltpu.sync_copy(u_idx_hbm.at[pl.ds(base, bpw)], uidx_v)
        pltpu.sync_copy(i_idx_hbm.at[pl.ds(base, bpw)], iidx_v)
        last_lane = lax.iota(jnp.int32, LANES) == (LANES - 1)

        def start_gathers(c):
            b = c % 2
            hu = pltpu.async_copy(
                user_hbm.at[uidx_v.at[pl.ds(c * CHUNK, CHUNK)]],
                u_rows.at[b], sems[2 * b])
            hi = pltpu.async_copy(
                item_hbm.at[iidx_v.at[pl.ds(c * CHUNK, CHUNK)]],
                i_rows.at[b], sems[2 * b + 1])
            return hu, hi

        handles = start_gathers(0)
        for c in range(nchunks):
            b = c % 2
            handles[0].wait()
            handles[1].wait()
            if c + 1 < nchunks:
                handles = start_gathers(c + 1)
            ub = u_rows.at[b]
            ib = i_rows.at[b]

            def group_body(g, _):
                for r in range(LANES):
                    row = g * LANES + r
                    p0 = ub[row, pl.ds(0, LANES)] * ib[row, pl.ds(0, LANES)]
                    p1 = ub[row, pl.ds(LANES, LANES)] * \
                        ib[row, pl.ds(LANES, LANES)]
                    for cc in range(2, DIM // LANES, 2):
                        p0 = p0 + ub[row, pl.ds(cc * LANES, LANES)] * \
                            ib[row, pl.ds(cc * LANES, LANES)]
                        p1 = p1 + ub[row, pl.ds((cc + 1) * LANES, LANES)] * \
                            ib[row, pl.ds((cc + 1) * LANES, LANES)]
                    csum = plsc.cumsum(p0 + p1)
                    plsc.store_scatter(
                        out_v,
                        [jnp.full((LANES,), c * CHUNK, jnp.int32) + row],
                        csum, mask=last_lane)
                return 0

            lax.fori_loop(0, CHUNK // LANES, group_body, 0)

        pltpu.sync_copy(out_v, out_hbm.at[pl.ds(base, bpw)])

    return two_tower


@jax.jit
def kernel(u_idx, i_idx, user_emb, item_emb):
    return _make_kernel(u_idx.shape[0])(u_idx, i_idx, user_emb, item_emb)
